# Initial kernel scaffold; baseline (speedup 1.0000x reference)
import jax
import jax.numpy as jnp
from jax.experimental import pallas as pl


def kernel(x, src1, dst1, src2, dst2, size1_tgt, size2_tgt, W1_l, b1_l, W1_r, W2_l, b2_l, W2_r):
    raise NotImplementedError("write your pallas kernel here")


# speedup vs baseline: 4.2492x; 4.2492x over previous
import functools

import jax
import jax.numpy as jnp
from jax import lax
from jax.experimental import pallas as pl
from jax.experimental.pallas import tpu as pltpu
from jax.experimental.pallas import tpu_sc as plsc

_NC = 2
_NS = 16
_NW = _NC * _NS
_CHUNK = 128


def _make_seg_sum(n_tab, d, s_pad, n_chunks):
    cpw = n_chunks // _NW
    rpt = s_pad // _NS
    assert n_chunks % _NW == 0 and s_pad % (16 * _NS) == 0 and d % 16 == 0
    mesh = plsc.VectorSubcoreMesh(core_axis_name="c", subcore_axis_name="s")

    @functools.partial(
        pl.kernel,
        out_type=[
            jax.ShapeDtypeStruct((_NC, s_pad, d), jnp.float32),
            jax.ShapeDtypeStruct((_NC * s_pad,), jnp.float32),
        ],
        mesh=mesh,
        scratch_types=[
            pltpu.VMEM((cpw, _CHUNK), jnp.int32),
            pltpu.VMEM((cpw, _CHUNK), jnp.int32),
            pltpu.VMEM((_CHUNK, d), jnp.float32),
            pltpu.VMEM((_CHUNK,), jnp.float32),
            pltpu.VMEM((16, d), jnp.float32),
            pltpu.VMEM((16,), jnp.float32),
            pltpu.VMEM((rpt,), jnp.float32),
            pltpu.VMEM_SHARED((s_pad, d), jnp.float32),
            pltpu.VMEM_SHARED((s_pad,), jnp.float32),
            pltpu.SemaphoreType.DMA,
        ],
    )
    def seg_sum(tab_hbm, src_hbm, dst_hbm, out_agg, out_cnt,
                src_v, dst_v, rows_v, ones_v, zrow_v, zcnt_v, cbounce_v,
                agg_sh, cnt_sh, sem):
        c = lax.axis_index("c")
        s = lax.axis_index("s")
        wid = c * _NS + s

        zero16 = jnp.zeros((16,), jnp.float32)
        for i in range(16):
            for j in range(d // 16):
                zrow_v[i, pl.ds(j * 16, 16)] = zero16
        zcnt_v[...] = zero16
        for j in range(_CHUNK // 16):
            ones_v[pl.ds(j * 16, 16)] = jnp.ones((16,), jnp.float32)

        for t in range(rpt // 16):
            pltpu.sync_copy(zrow_v, agg_sh.at[pl.ds(s * rpt + t * 16, 16)])
            pltpu.sync_copy(zcnt_v, cnt_sh.at[pl.ds(s * rpt + t * 16, 16)])

        pltpu.sync_copy(src_hbm.at[pl.ds(wid * cpw, cpw)], src_v)
        pltpu.sync_copy(dst_hbm.at[pl.ds(wid * cpw, cpw)], dst_v)

        plsc.subcore_barrier()

        def body(j, carry):
            pltpu.async_copy(tab_hbm.at[src_v.at[j]], rows_v, sem).wait()
            pltpu.sync_copy(rows_v, agg_sh.at[dst_v.at[j]], add=True)
            pltpu.sync_copy(ones_v, cnt_sh.at[dst_v.at[j]], add=True)
            return carry

        lax.fori_loop(0, cpw, body, 0)

        plsc.subcore_barrier()

        pltpu.sync_copy(agg_sh.at[pl.ds(s * rpt, rpt)],
                        out_agg.at[c, pl.ds(s * rpt, rpt)])
        pltpu.sync_copy(cnt_sh.at[pl.ds(s * rpt, rpt)], cbounce_v)
        pltpu.sync_copy(cbounce_v,
                        out_cnt.at[pl.ds(c * s_pad + s * rpt, rpt)])

    return seg_sum


def _tc1_body(agg_ref, cnt_ref, xt_ref, w1l_ref, b1l_ref, w1r_ref,
              w2r_ref, h_ref, r2_ref):
    agg = agg_ref[0, :2048, :] + agg_ref[1, :2048, :]
    cnt = cnt_ref[0, :2048] + cnt_ref[1, :2048]
    mean = agg / jnp.maximum(cnt, 1.0)[:, None]
    h = jnp.dot(mean, w1l_ref[...], preferred_element_type=jnp.float32)
    h = h + b1l_ref[...]
    h = h + jnp.dot(xt_ref[...], w1r_ref[...],
                    preferred_element_type=jnp.float32)
    h = jnp.maximum(h, 0.0)
    h_ref[...] = h
    r2_ref[...] = jnp.dot(h[:512], w2r_ref[...],
                          preferred_element_type=jnp.float32)


def _tc2_body(agg_ref, cnt_ref, r2_ref, w2l_ref, b2l_ref, out_ref):
    agg = agg_ref[0, :512, :] + agg_ref[1, :512, :]
    cnt = cnt_ref[0, :512] + cnt_ref[1, :512]
    mean = agg / jnp.maximum(cnt, 1.0)[:, None]
    z = jnp.dot(mean, w2l_ref[...], preferred_element_type=jnp.float32)
    z = z + b2l_ref[...] + r2_ref[...]
    m = jnp.max(z, axis=-1, keepdims=True)
    lse = jnp.log(jnp.sum(jnp.exp(z - m), axis=-1, keepdims=True))
    out_ref[...] = (z - m) - lse


def _pad_edges(src, dst, dummy_row):
    e = src.shape[0]
    block = _NW * 8 * _CHUNK
    e_pad = -(-e // block) * block
    if e_pad != e:
        pad = e_pad - e
        src = jnp.concatenate([src, jnp.zeros((pad,), jnp.int32)])
        dst = jnp.concatenate([dst, jnp.full((pad,), dummy_row, jnp.int32)])
    return src.reshape(-1, _CHUNK), dst.reshape(-1, _CHUNK)


def kernel(x, src1, dst1, src2, dst2, size1_tgt, size2_tgt,
           W1_l, b1_l, W1_r, W2_l, b2_l, W2_r):
    n1_tgt, n2_tgt = 2048, 512
    x = x.astype(jnp.float32)
    src1 = src1.astype(jnp.int32)
    dst1 = dst1.astype(jnp.int32)
    src2 = src2.astype(jnp.int32)
    dst2 = dst2.astype(jnp.int32)

    s1_pad = 2304
    src1p, dst1p = _pad_edges(src1, dst1, n1_tgt)
    agg1, cnt1 = _make_seg_sum(x.shape[0], x.shape[1], s1_pad,
                               src1p.shape[0])(x, src1p, dst1p)
    cnt1 = cnt1.reshape(_NC, s1_pad)

    h, r2 = pl.pallas_call(
        _tc1_body,
        out_shape=[
            jax.ShapeDtypeStruct((n1_tgt, W1_l.shape[1]), jnp.float32),
            jax.ShapeDtypeStruct((n2_tgt, W2_r.shape[1]), jnp.float32),
        ],
    )(agg1, cnt1, x[:n1_tgt], W1_l, b1_l.reshape(1, -1), W1_r, W2_r)

    s2_pad = 768
    src2p, dst2p = _pad_edges(src2, dst2, n2_tgt)
    agg2, cnt2 = _make_seg_sum(h.shape[0], h.shape[1], s2_pad,
                               src2p.shape[0])(h, src2p, dst2p)
    cnt2 = cnt2.reshape(_NC, s2_pad)

    out = pl.pallas_call(
        _tc2_body,
        out_shape=jax.ShapeDtypeStruct((n2_tgt, W2_l.shape[1]), jnp.float32),
    )(agg2, cnt2, r2, W2_l, b2_l.reshape(1, -1))
    return out


# speedup vs baseline: 4.8147x; 1.1331x over previous
import functools

import jax
import jax.numpy as jnp
from jax import lax
from jax.experimental import pallas as pl
from jax.experimental.pallas import tpu as pltpu
from jax.experimental.pallas import tpu_sc as plsc

_NC = 2
_NS = 16
_NW = _NC * _NS
_CHUNK = 128


def _make_seg_sum(n_tab, d, s_pad, n_chunks):
    cpw = n_chunks // _NW
    rpt = s_pad // _NS
    assert n_chunks % _NW == 0 and s_pad % (16 * _NS) == 0 and d % 16 == 0
    mesh = plsc.VectorSubcoreMesh(core_axis_name="c", subcore_axis_name="s")

    @functools.partial(
        pl.kernel,
        out_type=[
            jax.ShapeDtypeStruct((_NC, s_pad, d), jnp.float32),
            jax.ShapeDtypeStruct((_NC * s_pad,), jnp.float32),
        ],
        mesh=mesh,
        scratch_types=[
            pltpu.VMEM((cpw, _CHUNK), jnp.int32),
            pltpu.VMEM((cpw, _CHUNK), jnp.int32),
            pltpu.VMEM((2, _CHUNK, d), jnp.float32),
            pltpu.VMEM((_CHUNK,), jnp.float32),
            pltpu.VMEM((16, d), jnp.float32),
            pltpu.VMEM((16,), jnp.float32),
            pltpu.VMEM((rpt,), jnp.float32),
            pltpu.VMEM_SHARED((s_pad, d), jnp.float32),
            pltpu.VMEM_SHARED((s_pad,), jnp.float32),
            pltpu.SemaphoreType.DMA,
            pltpu.SemaphoreType.DMA,
        ],
    )
    def seg_sum(tab_hbm, src_hbm, dst_hbm, out_agg, out_cnt,
                src_v, dst_v, rows_v, ones_v, zrow_v, zcnt_v, cbounce_v,
                agg_sh, cnt_sh, sem0, sem1):
        c = lax.axis_index("c")
        s = lax.axis_index("s")
        wid = c * _NS + s

        zero16 = jnp.zeros((16,), jnp.float32)
        for i in range(16):
            for j in range(d // 16):
                zrow_v[i, pl.ds(j * 16, 16)] = zero16
        zcnt_v[...] = zero16
        for j in range(_CHUNK // 16):
            ones_v[pl.ds(j * 16, 16)] = jnp.ones((16,), jnp.float32)

        for t in range(rpt // 16):
            pltpu.sync_copy(zrow_v, agg_sh.at[pl.ds(s * rpt + t * 16, 16)])
            pltpu.sync_copy(zcnt_v, cnt_sh.at[pl.ds(s * rpt + t * 16, 16)])

        pltpu.sync_copy(src_hbm.at[pl.ds(wid * cpw, cpw)], src_v)
        pltpu.sync_copy(dst_hbm.at[pl.ds(wid * cpw, cpw)], dst_v)

        plsc.subcore_barrier()

        sems = (sem0, sem1)
        pltpu.async_copy(tab_hbm.at[src_v.at[0]], rows_v.at[0], sem0)
        pltpu.async_copy(tab_hbm.at[src_v.at[1]], rows_v.at[1], sem1)

        def body(i, carry):
            for b in range(2):
                j = 2 * i + b
                pltpu.make_async_copy(tab_hbm.at[src_v.at[j]],
                                      rows_v.at[b], sems[b]).wait()
                pltpu.sync_copy(rows_v.at[b], agg_sh.at[dst_v.at[j]],
                                add=True)
                pltpu.sync_copy(ones_v, cnt_sh.at[dst_v.at[j]], add=True)

                @pl.when(j + 2 < cpw)
                def _():
                    pltpu.async_copy(tab_hbm.at[src_v.at[j + 2]],
                                     rows_v.at[b], sems[b])
            return carry

        assert cpw % 2 == 0
        lax.fori_loop(0, cpw // 2, body, 0)

        plsc.subcore_barrier()

        pltpu.sync_copy(agg_sh.at[pl.ds(s * rpt, rpt)],
                        out_agg.at[c, pl.ds(s * rpt, rpt)])
        pltpu.sync_copy(cnt_sh.at[pl.ds(s * rpt, rpt)], cbounce_v)
        pltpu.sync_copy(cbounce_v,
                        out_cnt.at[pl.ds(c * s_pad + s * rpt, rpt)])

    return seg_sum


def _tc1_body(agg_ref, cnt_ref, xt_ref, w1l_ref, b1l_ref, w1r_ref,
              w2r_ref, h_ref, r2_ref):
    agg = agg_ref[0, :2048, :] + agg_ref[1, :2048, :]
    cnt = cnt_ref[0, :2048] + cnt_ref[1, :2048]
    mean = agg / jnp.maximum(cnt, 1.0)[:, None]
    h = jnp.dot(mean, w1l_ref[...], preferred_element_type=jnp.float32)
    h = h + b1l_ref[...]
    h = h + jnp.dot(xt_ref[...], w1r_ref[...],
                    preferred_element_type=jnp.float32)
    h = jnp.maximum(h, 0.0)
    h_ref[...] = h
    r2_ref[...] = jnp.dot(h[:512], w2r_ref[...],
                          preferred_element_type=jnp.float32)


def _tc2_body(agg_ref, cnt_ref, r2_ref, w2l_ref, b2l_ref, out_ref):
    agg = agg_ref[0, :512, :] + agg_ref[1, :512, :]
    cnt = cnt_ref[0, :512] + cnt_ref[1, :512]
    mean = agg / jnp.maximum(cnt, 1.0)[:, None]
    z = jnp.dot(mean, w2l_ref[...], preferred_element_type=jnp.float32)
    z = z + b2l_ref[...] + r2_ref[...]
    m = jnp.max(z, axis=-1, keepdims=True)
    lse = jnp.log(jnp.sum(jnp.exp(z - m), axis=-1, keepdims=True))
    out_ref[...] = (z - m) - lse


def _pad_edges(src, dst, dummy_row, s_pad):
    e = src.shape[0]
    block = _NW * 8 * _CHUNK
    e_pad = -(-e // block) * block
    if e_pad != e:
        pad = e_pad - e
        src = jnp.concatenate([src, jnp.zeros((pad,), jnp.int32)])
        dpad = dummy_row + jnp.arange(pad, dtype=jnp.int32) % (s_pad - dummy_row)
        dst = jnp.concatenate([dst, dpad])
    return src.reshape(-1, _CHUNK), dst.reshape(-1, _CHUNK)


def kernel(x, src1, dst1, src2, dst2, size1_tgt, size2_tgt,
           W1_l, b1_l, W1_r, W2_l, b2_l, W2_r):
    n1_tgt, n2_tgt = 2048, 512
    x = x.astype(jnp.float32)
    src1 = src1.astype(jnp.int32)
    dst1 = dst1.astype(jnp.int32)
    src2 = src2.astype(jnp.int32)
    dst2 = dst2.astype(jnp.int32)

    s1_pad = 2304
    src1p, dst1p = _pad_edges(src1, dst1, n1_tgt, s1_pad)
    agg1, cnt1 = _make_seg_sum(x.shape[0], x.shape[1], s1_pad,
                               src1p.shape[0])(x, src1p, dst1p)
    cnt1 = cnt1.reshape(_NC, s1_pad)

    h, r2 = pl.pallas_call(
        _tc1_body,
        out_shape=[
            jax.ShapeDtypeStruct((n1_tgt, W1_l.shape[1]), jnp.float32),
            jax.ShapeDtypeStruct((n2_tgt, W2_r.shape[1]), jnp.float32),
        ],
    )(agg1, cnt1, x[:n1_tgt], W1_l, b1_l.reshape(1, -1), W1_r, W2_r)

    s2_pad = 768
    src2p, dst2p = _pad_edges(src2, dst2, n2_tgt, s2_pad)
    agg2, cnt2 = _make_seg_sum(h.shape[0], h.shape[1], s2_pad,
                               src2p.shape[0])(h, src2p, dst2p)
    cnt2 = cnt2.reshape(_NC, s2_pad)

    out = pl.pallas_call(
        _tc2_body,
        out_shape=jax.ShapeDtypeStruct((n2_tgt, W2_l.shape[1]), jnp.float32),
    )(agg2, cnt2, r2, W2_l, b2_l.reshape(1, -1))
    return out


# speedup vs baseline: 5.1367x; 1.0669x over previous
import functools

import jax
import jax.numpy as jnp
from jax import lax
from jax.experimental import pallas as pl
from jax.experimental.pallas import tpu as pltpu
from jax.experimental.pallas import tpu_sc as plsc

_NC = 2
_NS = 16
_NW = _NC * _NS
_CHUNK = 128


def _make_seg_sum(n_tab, d, s_pad, n_chunks, stage_table):
    cpw = n_chunks // _NW
    rpt = s_pad // _NS
    tpt = n_tab // _NS
    assert n_chunks % _NW == 0 and s_pad % (16 * _NS) == 0 and d % 16 == 0
    assert n_tab % (8 * _NS) == 0
    mesh = plsc.VectorSubcoreMesh(core_axis_name="c", subcore_axis_name="s")

    @functools.partial(
        pl.kernel,
        out_type=[
            jax.ShapeDtypeStruct((_NC, s_pad, d), jnp.float32),
            jax.ShapeDtypeStruct((_NC * s_pad,), jnp.float32),
        ],
        mesh=mesh,
        scratch_types=[
            pltpu.VMEM((cpw, _CHUNK), jnp.int32),
            pltpu.VMEM((cpw, _CHUNK), jnp.int32),
            pltpu.VMEM((4, _CHUNK, d), jnp.float32),
            pltpu.VMEM((_CHUNK,), jnp.float32),
            pltpu.VMEM((16, d), jnp.float32),
            pltpu.VMEM((16,), jnp.float32),
            pltpu.VMEM((rpt,), jnp.float32),
        ] + ([pltpu.VMEM_SHARED((n_tab, d), jnp.float32)] if stage_table
             else []) + [
            pltpu.VMEM_SHARED((s_pad, d), jnp.float32),
            pltpu.VMEM_SHARED((s_pad,), jnp.float32),
            pltpu.SemaphoreType.DMA,
            pltpu.SemaphoreType.DMA,
            pltpu.SemaphoreType.DMA,
            pltpu.SemaphoreType.DMA,
        ],
    )
    def seg_sum(tab_hbm, src_hbm, dst_hbm, out_agg, out_cnt,
                src_v, dst_v, rows_v, ones_v, zrow_v, zcnt_v, cbounce_v,
                *rest):
        if stage_table:
            tab_sh, agg_sh, cnt_sh = rest[:3]
            sems = rest[3:]
        else:
            agg_sh, cnt_sh = rest[:2]
            sems = rest[2:]
            tab_sh = None
        c = lax.axis_index("c")
        s = lax.axis_index("s")
        wid = c * _NS + s
        gsrc = tab_sh if stage_table else tab_hbm

        zero16 = jnp.zeros((16,), jnp.float32)
        for i in range(16):
            for j in range(d // 16):
                zrow_v[i, pl.ds(j * 16, 16)] = zero16
        zcnt_v[...] = zero16
        for j in range(_CHUNK // 16):
            ones_v[pl.ds(j * 16, 16)] = jnp.ones((16,), jnp.float32)

        for t in range(rpt // 16):
            pltpu.sync_copy(zrow_v, agg_sh.at[pl.ds(s * rpt + t * 16, 16)])
            pltpu.sync_copy(zcnt_v, cnt_sh.at[pl.ds(s * rpt + t * 16, 16)])

        pltpu.sync_copy(src_hbm.at[pl.ds(wid * cpw, cpw)], src_v)
        pltpu.sync_copy(dst_hbm.at[pl.ds(wid * cpw, cpw)], dst_v)
        if stage_table:
            pltpu.sync_copy(tab_hbm.at[pl.ds(s * tpt, tpt)],
                            tab_sh.at[pl.ds(s * tpt, tpt)])

        plsc.subcore_barrier()

        nbuf = 4
        for b in range(nbuf):
            pltpu.async_copy(gsrc.at[src_v.at[b]], rows_v.at[b], sems[b])

        def body(i, carry):
            for b in range(nbuf):
                j = nbuf * i + b
                pltpu.make_async_copy(gsrc.at[src_v.at[j]],
                                      rows_v.at[b], sems[b]).wait()
                pltpu.sync_copy(rows_v.at[b], agg_sh.at[dst_v.at[j]],
                                add=True)
                pltpu.sync_copy(ones_v, cnt_sh.at[dst_v.at[j]], add=True)

                @pl.when(j + nbuf < cpw)
                def _():
                    pltpu.async_copy(gsrc.at[src_v.at[j + nbuf]],
                                     rows_v.at[b], sems[b])
            return carry

        assert cpw % nbuf == 0
        lax.fori_loop(0, cpw // nbuf, body, 0)

        plsc.subcore_barrier()

        pltpu.sync_copy(agg_sh.at[pl.ds(s * rpt, rpt)],
                        out_agg.at[c, pl.ds(s * rpt, rpt)])
        pltpu.sync_copy(cnt_sh.at[pl.ds(s * rpt, rpt)], cbounce_v)
        pltpu.sync_copy(cbounce_v,
                        out_cnt.at[pl.ds(c * s_pad + s * rpt, rpt)])

    return seg_sum


def _tc1_body(agg_ref, cnt_ref, xt_ref, w1l_ref, b1l_ref, w1r_ref,
              w2r_ref, h_ref, r2_ref):
    agg = agg_ref[0, :2048, :] + agg_ref[1, :2048, :]
    cnt = cnt_ref[0, :2048] + cnt_ref[1, :2048]
    mean = agg / jnp.maximum(cnt, 1.0)[:, None]
    h = jnp.dot(mean, w1l_ref[...], preferred_element_type=jnp.float32)
    h = h + b1l_ref[...]
    h = h + jnp.dot(xt_ref[...], w1r_ref[...],
                    preferred_element_type=jnp.float32)
    h = jnp.maximum(h, 0.0)
    h_ref[...] = h
    r2_ref[...] = jnp.dot(h[:512], w2r_ref[...],
                          preferred_element_type=jnp.float32)


def _tc2_body(agg_ref, cnt_ref, r2_ref, w2l_ref, b2l_ref, out_ref):
    agg = agg_ref[0, :512, :] + agg_ref[1, :512, :]
    cnt = cnt_ref[0, :512] + cnt_ref[1, :512]
    mean = agg / jnp.maximum(cnt, 1.0)[:, None]
    z = jnp.dot(mean, w2l_ref[...], preferred_element_type=jnp.float32)
    z = z + b2l_ref[...] + r2_ref[...]
    m = jnp.max(z, axis=-1, keepdims=True)
    lse = jnp.log(jnp.sum(jnp.exp(z - m), axis=-1, keepdims=True))
    out_ref[...] = (z - m) - lse


def _pad_edges(src, dst, dummy_row, s_pad):
    e = src.shape[0]
    block = _NW * 8 * _CHUNK
    e_pad = -(-e // block) * block
    if e_pad != e:
        pad = e_pad - e
        src = jnp.concatenate([src, jnp.zeros((pad,), jnp.int32)])
        dpad = dummy_row + jnp.arange(pad, dtype=jnp.int32) % (s_pad - dummy_row)
        dst = jnp.concatenate([dst, dpad])
    return src.reshape(-1, _CHUNK), dst.reshape(-1, _CHUNK)


def kernel(x, src1, dst1, src2, dst2, size1_tgt, size2_tgt,
           W1_l, b1_l, W1_r, W2_l, b2_l, W2_r):
    n1_tgt, n2_tgt = 2048, 512
    x = x.astype(jnp.float32)
    src1 = src1.astype(jnp.int32)
    dst1 = dst1.astype(jnp.int32)
    src2 = src2.astype(jnp.int32)
    dst2 = dst2.astype(jnp.int32)

    s1_pad = 2304
    src1p, dst1p = _pad_edges(src1, dst1, n1_tgt, s1_pad)
    n_tab1 = -(-x.shape[0] // 128) * 128
    if n_tab1 != x.shape[0]:
        x_tab = jnp.concatenate(
            [x, jnp.zeros((n_tab1 - x.shape[0], x.shape[1]), jnp.float32)])
    else:
        x_tab = x
    agg1, cnt1 = _make_seg_sum(n_tab1, x.shape[1], s1_pad,
                               src1p.shape[0], False)(x_tab, src1p, dst1p)
    cnt1 = cnt1.reshape(_NC, s1_pad)

    h, r2 = pl.pallas_call(
        _tc1_body,
        out_shape=[
            jax.ShapeDtypeStruct((n1_tgt, W1_l.shape[1]), jnp.float32),
            jax.ShapeDtypeStruct((n2_tgt, W2_r.shape[1]), jnp.float32),
        ],
    )(agg1, cnt1, x[:n1_tgt], W1_l, b1_l.reshape(1, -1), W1_r, W2_r)

    s2_pad = 768
    src2p, dst2p = _pad_edges(src2, dst2, n2_tgt, s2_pad)
    agg2, cnt2 = _make_seg_sum(h.shape[0], h.shape[1], s2_pad,
                               src2p.shape[0], False)(h, src2p, dst2p)
    cnt2 = cnt2.reshape(_NC, s2_pad)

    out = pl.pallas_call(
        _tc2_body,
        out_shape=jax.ShapeDtypeStruct((n2_tgt, W2_l.shape[1]), jnp.float32),
    )(agg2, cnt2, r2, W2_l, b2_l.reshape(1, -1))
    return out
